# baseline (speedup 1.0000x reference)
import functools

import jax
import jax.numpy as jnp
from jax import lax
from jax.experimental import pallas as pl
from jax.experimental.pallas import tpu as pltpu
from jax.experimental.pallas import tpu_sc as plsc

IN_F = 1000000
OUT_F = 32
BATCH = 16384
PACK = 4
WIDE = PACK * OUT_F

NC = 2
NS = 16
L = 16
NW = NC * NS
BPW = BATCH // NW
CHUNK = 128
NCHUNK = BPW // CHUNK
RSTEP = 16

_mesh = plsc.VectorSubcoreMesh(core_axis_name="c", subcore_axis_name="s")


@functools.partial(
    pl.kernel,
    mesh=_mesh,
    compiler_params=pltpu.CompilerParams(needs_layout_passes=False),
    out_type=jax.ShapeDtypeStruct((BATCH // PACK, WIDE), jnp.float32),
    scratch_types=[
        pltpu.VMEM((BPW,), jnp.int32),
        pltpu.VMEM((NCHUNK, CHUNK), jnp.int32),
        pltpu.VMEM((CHUNK, WIDE), jnp.float32),
        pltpu.VMEM((CHUNK, WIDE), jnp.float32),
        pltpu.VMEM((BPW // PACK, WIDE), jnp.float32),
        pltpu.VMEM((OUT_F,), jnp.float32),
        pltpu.SemaphoreType.DMA,
        pltpu.SemaphoreType.DMA,
        pltpu.SemaphoreType.DMA,
    ],
)
def _gather_bias(idx_hbm, table_hbm, bias_hbm, out_hbm,
                 idx_v, idx4_v, buf_a, buf_b, out_v, bias_v,
                 sem_a, sem_b, sem_o):
    wid = lax.axis_index("s") * NC + lax.axis_index("c")
    pltpu.sync_copy(idx_hbm.at[pl.ds(wid * BPW, BPW)], idx_v)
    pltpu.sync_copy(bias_hbm, bias_v)
    b0 = bias_v[pl.ds(0, L)]
    b1 = bias_v[pl.ds(L, L)]
    for j in range(NCHUNK):
        for k in range(CHUNK // L):
            idx4_v[j, pl.ds(k * L, L)] = (
                idx_v[pl.ds(j * CHUNK + k * L, L)] >> 2
            )
    bufs = [buf_a, buf_b]
    sems = [sem_a, sem_b]
    cps = [None] * NCHUNK
    out_cps = []
    cps[0] = pltpu.async_copy(table_hbm.at[idx4_v.at[0]], buf_a, sem_a)
    cps[1] = pltpu.async_copy(table_hbm.at[idx4_v.at[1]], buf_b, sem_b)
    for j in range(NCHUNK):
        buf = bufs[j % 2]
        cps[j].wait()

        def extract(i, carry, j=j, buf=buf):
            idx16 = idx_v[pl.ds(j * CHUNK + i * RSTEP, RSTEP)]
            off16 = (idx16 & 3) * OUT_F
            for k in range(RSTEP):
                r = i * RSTEP + k
                off = off16[k]
                v0 = buf[r, pl.ds(off, L)] + b0
                v1 = buf[r, pl.ds(off + L, L)] + b1
                ro = j * (CHUNK // PACK) + (RSTEP // PACK) * i + (k >> 2)
                co = (k & 3) * OUT_F
                out_v[ro, pl.ds(co, L)] = v0
                out_v[ro, pl.ds(co + L, L)] = v1
            return carry

        lax.fori_loop(0, CHUNK // RSTEP, extract, 0)
        if j + 2 < NCHUNK:
            cps[j + 2] = pltpu.async_copy(
                table_hbm.at[idx4_v.at[j + 2]], buf, sems[j % 2]
            )
        out_cps.append(pltpu.async_copy(
            out_v.at[pl.ds(j * (CHUNK // PACK), CHUNK // PACK)],
            out_hbm.at[pl.ds(wid * (BPW // PACK) + j * (CHUNK // PACK),
                             CHUNK // PACK)],
            sem_o,
        ))
    for c in out_cps:
        c.wait()


def kernel(input, weight, bias):
    idx = input.astype(jnp.int32)
    w128 = weight.reshape(IN_F // PACK, WIDE)
    out = _gather_bias(idx, w128, bias)
    return out.reshape(BATCH, OUT_F)
